# baseline (speedup 1.0000x reference)
import functools

import jax
import jax.numpy as jnp
from jax import lax
from jax.experimental import pallas as pl
from jax.experimental.pallas import tpu as pltpu
from jax.experimental.pallas import tpu_sc as plsc

N_TOTAL = 2097152
N_BINS_OUT = 10
NC = 2
NS = 16
L = 16
NW = NC * NS
PER_W = N_TOTAL // NW
BLK = 8192
NBLK = PER_W // BLK
HBINS = 11
HWORDS = HBINS * L

_mesh = plsc.VectorSubcoreMesh(core_axis_name="c", subcore_axis_name="s")


@functools.partial(
    pl.kernel,
    mesh=_mesh,
    out_type=(
        jax.ShapeDtypeStruct((NW, HWORDS), jnp.int32),
        jax.ShapeDtypeStruct((NW, HWORDS), jnp.int32),
        jax.ShapeDtypeStruct((NW, HWORDS), jnp.float32),
    ),
    scratch_types=[
        pltpu.VMEM((BLK, 2), jnp.float32),
        pltpu.VMEM((BLK,), jnp.int32),
        pltpu.VMEM((HWORDS,), jnp.int32),
        pltpu.VMEM((HWORDS,), jnp.int32),
        pltpu.VMEM((HWORDS,), jnp.float32),
    ],
    compiler_params=pltpu.CompilerParams(
        needs_layout_passes=False, use_tc_tiling_on_sc=False),
)
def _ece_hist(lg_hbm, lb_hbm, cnt_out, lab_out, prd_out,
              lg_v, lb_v, cnt_v, lab_v, prd_v):
    wid = lax.axis_index("s") * NC + lax.axis_index("c")

    lane = lax.iota(jnp.int32, L)
    ones_i = jnp.ones((L,), jnp.int32)
    zeros_idx = jnp.zeros((L,), jnp.int32)
    z_i = jnp.zeros((L,), jnp.int32)
    z_f = jnp.zeros((L,), jnp.float32)

    for b in range(HBINS):
        cnt_v[pl.ds(b * L, L)] = z_i
        lab_v[pl.ds(b * L, L)] = z_i
        prd_v[pl.ds(b * L, L)] = z_f

    elem0 = wid * PER_W

    def do_block(blk):
        off = elem0 + blk * BLK
        pltpu.sync_copy(lg_hbm.at[pl.ds(off, BLK)], lg_v)
        pltpu.sync_copy(lb_hbm.at[pl.ds(off, BLK)], lb_v)

        def body(j, carry):
            rows = lane + j * L
            l0 = plsc.load_gather(lg_v, [rows, zeros_idx])
            l1 = plsc.load_gather(lg_v, [rows, ones_i])
            lb16 = lb_v[pl.ds(j * L, L)]
            e = jnp.exp(l0 - l1)
            p = 1.0 / (1.0 + e)
            bin_ = (p * 10.0).astype(jnp.int32)
            addr = bin_ * L + lane
            plsc.addupdate_scatter(cnt_v, [addr], ones_i)
            plsc.addupdate_scatter(lab_v, [addr], lb16)
            plsc.addupdate_scatter(prd_v, [addr], p)
            return carry

        lax.fori_loop(0, BLK // L, body, 0)

    for blk in range(NBLK):
        do_block(blk)

    pltpu.sync_copy(cnt_v, cnt_out.at[wid])
    pltpu.sync_copy(lab_v, lab_out.at[wid])
    pltpu.sync_copy(prd_v, prd_out.at[wid])


def kernel(logits, labels):
    cnt, lab, prd = _ece_hist(logits, labels)
    sizes = cnt.reshape(NW, HBINS, L).sum(axis=(0, 2))[:N_BINS_OUT]
    lab_s = lab.reshape(NW, HBINS, L).sum(axis=(0, 2))[:N_BINS_OUT]
    prd_s = prd.reshape(NW, HBINS, L).sum(axis=(0, 2))[:N_BINS_OUT]
    sizes = sizes.astype(jnp.float32)
    lab_s = lab_s.astype(jnp.float32)
    nonempty = sizes > 0
    safe = jnp.where(nonempty, sizes, 1.0)
    accs = jnp.where(nonempty, lab_s / safe, 0.0)
    confs = jnp.where(nonempty, prd_s / safe, 0.0)
    return jnp.sum(sizes / jnp.sum(sizes) * jnp.abs(accs - confs))


# speedup vs baseline: 22.4284x; 22.4284x over previous
import functools

import jax
import jax.numpy as jnp
from jax import lax
from jax.experimental import pallas as pl
from jax.experimental.pallas import tpu as pltpu
from jax.experimental.pallas import tpu_sc as plsc

N_TOTAL = 2097152
N_BINS_OUT = 10
NC = 2
NS = 16
L = 16
NW = NC * NS
PER_W = N_TOTAL // NW
BLK = 8192
NBLK = PER_W // BLK
TILE = 128
NTILE = BLK // TILE
HBINS = 11
HWORDS = HBINS * L
NREP = 2
HTOT = NREP * HWORDS

_mesh = plsc.VectorSubcoreMesh(core_axis_name="c", subcore_axis_name="s")


@functools.partial(
    pl.kernel,
    mesh=_mesh,
    out_type=(
        jax.ShapeDtypeStruct((NW, HTOT), jnp.int32),
        jax.ShapeDtypeStruct((NW, HTOT), jnp.int32),
        jax.ShapeDtypeStruct((NW, HTOT), jnp.float32),
    ),
    scratch_types=[
        pltpu.VMEM((2 * BLK,), jnp.float32),
        pltpu.VMEM((BLK,), jnp.int32),
        pltpu.VMEM((HTOT,), jnp.int32),
        pltpu.VMEM((HTOT,), jnp.int32),
        pltpu.VMEM((HTOT,), jnp.float32),
    ],
    compiler_params=pltpu.CompilerParams(
        needs_layout_passes=False, use_tc_tiling_on_sc=False),
)
def _ece_hist(lg_hbm, lb_hbm, cnt_out, lab_out, prd_out,
              lg_v, lb_v, cnt_v, lab_v, prd_v):
    wid = lax.axis_index("s") * NC + lax.axis_index("c")

    lane = lax.iota(jnp.int32, L)
    ones_i = jnp.ones((L,), jnp.int32)
    z_i = jnp.zeros((L,), jnp.int32)
    z_f = jnp.zeros((L,), jnp.float32)

    for b in range(NREP * HBINS):
        cnt_v[pl.ds(b * L, L)] = z_i
        lab_v[pl.ds(b * L, L)] = z_i
        prd_v[pl.ds(b * L, L)] = z_f

    elem0 = wid * PER_W

    def do_block(blk):
        off = elem0 + blk * BLK
        pltpu.sync_copy(lg_hbm.at[pl.ds(off * 2, 2 * BLK)], lg_v)
        pltpu.sync_copy(lb_hbm.at[pl.ds(off, BLK)], lb_v)

        def body(t, carry):
            base = t * (2 * TILE)
            lbase = t * TILE
            for i in range(TILE // L):
                l0 = lg_v[pl.ds(base + i * L, L)]
                l1 = lg_v[pl.ds(base + TILE + i * L, L)]
                lb16 = lb_v[pl.ds(lbase + i * L, L)]
                e = jnp.exp(l0 - l1)
                p = 1.0 / (1.0 + e)
                bin_ = (p * 10.0).astype(jnp.int32)
                addr = bin_ * L + lane + (i % NREP) * HWORDS
                plsc.addupdate_scatter(cnt_v, [addr], ones_i)
                plsc.addupdate_scatter(lab_v, [addr], lb16)
                plsc.addupdate_scatter(prd_v, [addr], p)
            return carry

        lax.fori_loop(0, NTILE, body, 0)

    for blk in range(NBLK):
        do_block(blk)

    pltpu.sync_copy(cnt_v, cnt_out.at[wid])
    pltpu.sync_copy(lab_v, lab_out.at[wid])
    pltpu.sync_copy(prd_v, prd_out.at[wid])


def kernel(logits, labels):
    lg_flat = (logits.reshape(N_TOTAL // TILE, TILE, 2)
               .transpose(0, 2, 1).reshape(-1))
    cnt, lab, prd = _ece_hist(lg_flat, labels)
    sizes = cnt.reshape(NW * NREP, HBINS, L).sum(axis=(0, 2))[:N_BINS_OUT]
    lab_s = lab.reshape(NW * NREP, HBINS, L).sum(axis=(0, 2))[:N_BINS_OUT]
    prd_s = prd.reshape(NW * NREP, HBINS, L).sum(axis=(0, 2))[:N_BINS_OUT]
    sizes = sizes.astype(jnp.float32)
    lab_s = lab_s.astype(jnp.float32)
    nonempty = sizes > 0
    safe = jnp.where(nonempty, sizes, 1.0)
    accs = jnp.where(nonempty, lab_s / safe, 0.0)
    confs = jnp.where(nonempty, prd_s / safe, 0.0)
    return jnp.sum(sizes / jnp.sum(sizes) * jnp.abs(accs - confs))


# speedup vs baseline: 57.0876x; 2.5453x over previous
import functools

import jax
import jax.numpy as jnp
from jax import lax
from jax.experimental import pallas as pl
from jax.experimental.pallas import tpu as pltpu
from jax.experimental.pallas import tpu_sc as plsc

N_TOTAL = 2097152
N_BINS_OUT = 10
NC = 2
NS = 16
L = 16
NW = NC * NS
PER_W = N_TOTAL // NW
BLK = 8192
NBLK = PER_W // BLK
TILE = 128
NTILE = BLK // TILE
HBINS = 11
HWORDS = HBINS * L
NREP = 2
HTOT = NREP * HWORDS

_mesh = plsc.VectorSubcoreMesh(core_axis_name="c", subcore_axis_name="s")


@functools.partial(
    pl.kernel,
    mesh=_mesh,
    out_type=(
        jax.ShapeDtypeStruct((NW, HTOT), jnp.int32),
        jax.ShapeDtypeStruct((NW, HTOT), jnp.int32),
        jax.ShapeDtypeStruct((NW, HTOT), jnp.float32),
    ),
    scratch_types=[
        pltpu.VMEM((2 * BLK,), jnp.float32),
        pltpu.VMEM((BLK,), jnp.int32),
        pltpu.VMEM((HTOT,), jnp.int32),
        pltpu.VMEM((HTOT,), jnp.int32),
        pltpu.VMEM((HTOT,), jnp.float32),
    ],
    compiler_params=pltpu.CompilerParams(
        needs_layout_passes=False, use_tc_tiling_on_sc=False),
)
def _ece_hist(lg_hbm, lb_hbm, cnt_out, lab_out, prd_out,
              lg_v, lb_v, cnt_v, lab_v, prd_v):
    wid = lax.axis_index("s") * NC + lax.axis_index("c")

    lane = lax.iota(jnp.int32, L)
    ones_i = jnp.ones((L,), jnp.int32)
    z_i = jnp.zeros((L,), jnp.int32)
    z_f = jnp.zeros((L,), jnp.float32)

    for b in range(NREP * HBINS):
        cnt_v[pl.ds(b * L, L)] = z_i
        lab_v[pl.ds(b * L, L)] = z_i
        prd_v[pl.ds(b * L, L)] = z_f

    elem0 = wid * PER_W

    def do_block(blk, bcarry):
        off = elem0 + blk * BLK
        pltpu.sync_copy(lg_hbm.at[pl.ds(off * 2, 2 * BLK)], lg_v)
        pltpu.sync_copy(lb_hbm.at[pl.ds(off, BLK)], lb_v)

        def body(t):
            base = t * (2 * TILE)
            lbase = t * TILE
            for i in range(TILE // L):
                l0 = lg_v[pl.ds(base + i * L, L)]
                l1 = lg_v[pl.ds(base + TILE + i * L, L)]
                lb16 = lb_v[pl.ds(lbase + i * L, L)]
                e = jnp.exp(l0 - l1)
                p = 1.0 / (1.0 + e)
                bin_ = (p * 10.0).astype(jnp.int32)
                addr = bin_ * L + lane + (i % NREP) * HWORDS
                plsc.addupdate_scatter(cnt_v, [addr], ones_i)
                plsc.addupdate_scatter(lab_v, [addr], lb16)
                plsc.addupdate_scatter(prd_v, [addr], p)

        plsc.parallel_loop(0, NTILE)(body)
        return bcarry

    lax.fori_loop(0, NBLK, do_block, 0)

    pltpu.sync_copy(cnt_v, cnt_out.at[wid])
    pltpu.sync_copy(lab_v, lab_out.at[wid])
    pltpu.sync_copy(prd_v, prd_out.at[wid])


def kernel(logits, labels):
    lg_flat = (logits.reshape(N_TOTAL // TILE, TILE, 2)
               .transpose(0, 2, 1).reshape(-1))
    cnt, lab, prd = _ece_hist(lg_flat, labels)
    sizes = cnt.reshape(NW * NREP, HBINS, L).sum(axis=(0, 2))[:N_BINS_OUT]
    lab_s = lab.reshape(NW * NREP, HBINS, L).sum(axis=(0, 2))[:N_BINS_OUT]
    prd_s = prd.reshape(NW * NREP, HBINS, L).sum(axis=(0, 2))[:N_BINS_OUT]
    sizes = sizes.astype(jnp.float32)
    lab_s = lab_s.astype(jnp.float32)
    nonempty = sizes > 0
    safe = jnp.where(nonempty, sizes, 1.0)
    accs = jnp.where(nonempty, lab_s / safe, 0.0)
    confs = jnp.where(nonempty, prd_s / safe, 0.0)
    return jnp.sum(sizes / jnp.sum(sizes) * jnp.abs(accs - confs))


# speedup vs baseline: 68.6938x; 1.2033x over previous
import functools

import jax
import jax.numpy as jnp
from jax import lax
from jax.experimental import pallas as pl
from jax.experimental.pallas import tpu as pltpu
from jax.experimental.pallas import tpu_sc as plsc

N_TOTAL = 2097152
N_BINS_OUT = 10
NC = 2
NS = 16
L = 16
NW = NC * NS
PER_W = N_TOTAL // NW
BLK = 8192
NBLK = PER_W // BLK
TILE = 128
NTILE = BLK // TILE
HBINS = 11
HWORDS = HBINS * L
NREP = TILE // L
HTOT = NREP * HWORDS

_mesh = plsc.VectorSubcoreMesh(core_axis_name="c", subcore_axis_name="s")


@functools.partial(
    pl.kernel,
    mesh=_mesh,
    out_type=(
        jax.ShapeDtypeStruct((NW, HTOT), jnp.int32),
        jax.ShapeDtypeStruct((NW, HTOT), jnp.int32),
        jax.ShapeDtypeStruct((NW, HTOT), jnp.float32),
    ),
    scratch_types=[
        pltpu.VMEM((2 * BLK,), jnp.float32),
        pltpu.VMEM((2 * BLK,), jnp.float32),
        pltpu.VMEM((BLK,), jnp.int32),
        pltpu.VMEM((BLK,), jnp.int32),
        pltpu.VMEM((HTOT,), jnp.int32),
        pltpu.VMEM((HTOT,), jnp.int32),
        pltpu.VMEM((HTOT,), jnp.float32),
        pltpu.SemaphoreType.DMA,
        pltpu.SemaphoreType.DMA,
    ],
    compiler_params=pltpu.CompilerParams(
        needs_layout_passes=False, use_tc_tiling_on_sc=False),
)
def _ece_hist(lg_hbm, lb_hbm, cnt_out, lab_out, prd_out,
              lg_a, lg_b, lb_a, lb_b, cnt_v, lab_v, prd_v, sem_a, sem_b):
    wid = lax.axis_index("s") * NC + lax.axis_index("c")

    lane = lax.iota(jnp.int32, L)
    ones_i = jnp.ones((L,), jnp.int32)
    z_i = jnp.zeros((L,), jnp.int32)
    z_f = jnp.zeros((L,), jnp.float32)

    for b in range(NREP * HBINS):
        cnt_v[pl.ds(b * L, L)] = z_i
        lab_v[pl.ds(b * L, L)] = z_i
        prd_v[pl.ds(b * L, L)] = z_f

    elem0 = wid * PER_W

    def start_blk(blk, lgbuf, lbbuf, sem):
        off = elem0 + blk * BLK
        pltpu.async_copy(lg_hbm.at[pl.ds(off * 2, 2 * BLK)], lgbuf, sem)
        pltpu.async_copy(lb_hbm.at[pl.ds(off, BLK)], lbbuf, sem)

    def wait_blk(lgbuf, lbbuf, sem):
        pltpu.make_async_copy(lg_hbm.at[pl.ds(0, 2 * BLK)], lgbuf, sem).wait()
        pltpu.make_async_copy(lb_hbm.at[pl.ds(0, BLK)], lbbuf, sem).wait()

    def compute(lg_v, lb_v):
        def body(t):
            base = t * (2 * TILE)
            lbase = t * TILE
            for i in range(TILE // L):
                l0 = lg_v[pl.ds(base + i * L, L)]
                l1 = lg_v[pl.ds(base + TILE + i * L, L)]
                lb16 = lb_v[pl.ds(lbase + i * L, L)]
                e = jnp.exp(l0 - l1)
                p = 1.0 / (1.0 + e)
                bin_ = (p * 10.0).astype(jnp.int32)
                addr = bin_ * L + lane
                rep = pl.ds(i * HWORDS, HWORDS)
                plsc.addupdate_scatter(cnt_v.at[rep], [addr], ones_i)
                plsc.addupdate_scatter(lab_v.at[rep], [addr], lb16)
                plsc.addupdate_scatter(prd_v.at[rep], [addr], p)

        plsc.parallel_loop(0, NTILE)(body)

    start_blk(0, lg_a, lb_a, sem_a)

    def super_body(k, c):
        blk_a = 2 * k
        wait_blk(lg_a, lb_a, sem_a)
        start_blk(blk_a + 1, lg_b, lb_b, sem_b)
        compute(lg_a, lb_a)
        wait_blk(lg_b, lb_b, sem_b)

        @pl.when(k < NBLK // 2 - 1)
        def _():
            start_blk(blk_a + 2, lg_a, lb_a, sem_a)

        compute(lg_b, lb_b)
        return c

    lax.fori_loop(0, NBLK // 2, super_body, 0)

    pltpu.sync_copy(cnt_v, cnt_out.at[wid])
    pltpu.sync_copy(lab_v, lab_out.at[wid])
    pltpu.sync_copy(prd_v, prd_out.at[wid])


def kernel(logits, labels):
    lg_flat = (logits.reshape(N_TOTAL // TILE, TILE, 2)
               .transpose(0, 2, 1).reshape(-1))
    cnt, lab, prd = _ece_hist(lg_flat, labels)
    sizes = cnt.reshape(NW * NREP, HBINS, L).sum(axis=(0, 2))[:N_BINS_OUT]
    lab_s = lab.reshape(NW * NREP, HBINS, L).sum(axis=(0, 2))[:N_BINS_OUT]
    prd_s = prd.reshape(NW * NREP, HBINS, L).sum(axis=(0, 2))[:N_BINS_OUT]
    sizes = sizes.astype(jnp.float32)
    lab_s = lab_s.astype(jnp.float32)
    nonempty = sizes > 0
    safe = jnp.where(nonempty, sizes, 1.0)
    accs = jnp.where(nonempty, lab_s / safe, 0.0)
    confs = jnp.where(nonempty, prd_s / safe, 0.0)
    return jnp.sum(sizes / jnp.sum(sizes) * jnp.abs(accs - confs))
